# baseline (speedup 1.0000x reference)
import functools

import jax
import jax.numpy as jnp
from jax import lax
from jax.experimental import pallas as pl
from jax.experimental.pallas import tpu as pltpu
from jax.experimental.pallas import tpu_sc as plsc

N = 10000
E = 160000
D = 300
DP = 304
NP = 10240
NT = 32
RPT = 320
CE = 320
NCHUNK = E // CE
GB = 16
KL = 352
SW = 16
EPT = E // NT
BLK = 1024

_MESH = plsc.VectorSubcoreMesh(core_axis_name="c", subcore_axis_name="s")
_SC_PARAMS = pltpu.CompilerParams(
    needs_layout_passes=False, use_tc_tiling_on_sc=False)


def _worker_id():
    return lax.axis_index("s") * 2 + lax.axis_index("c")


def _hist_body(row_hbm, hist_hbm, hl, st):
    w = _worker_id()
    zero16 = jnp.zeros((16,), jnp.float32)
    ones16 = jnp.ones((16,), jnp.float32)
    iota = lax.iota(jnp.int32, 16)

    @pl.loop(0, NP // 16)
    def _zero(i):
        hl[pl.ds(i * 16, 16)] = zero16

    base = w * EPT
    pltpu.sync_copy(row_hbm.at[pl.ds(base, EPT)], st)

    nfull = EPT // 16

    def step(i, carry):
        idxv = st[pl.ds(i * 16, 16)]
        for k in range(16):
            plsc.addupdate_scatter(hl, [idxv], ones16, mask=iota == k)
        return carry

    lax.fori_loop(0, nfull, step, 0)
    tailv = st[pl.ds(EPT - 16, 16)]
    for k in range(16 - (EPT - nfull * 16), 16):
        plsc.addupdate_scatter(hl, [tailv], ones16, mask=iota == k)
    pltpu.sync_copy(hl, hist_hbm.at[w])


def _hist_call(row):
    f = pl.kernel(
        _hist_body,
        out_type=jax.ShapeDtypeStruct((NT, NP), jnp.float32),
        mesh=_MESH,
        scratch_types=[
            pltpu.VMEM((NP,), jnp.float32),
            pltpu.VMEM((EPT,), jnp.int32),
        ],
        compiler_params=_SC_PARAMS,
    )
    return f(row)


def _edges_body(with_s, npar, *refs):
    if with_s:
        (g_hbm, row_hbm, col_hbm, code_hbm, p_hbm, s_hbm,
         st_row, st_col, st_code, krow, kdst, kcode, sl, ssem,
         kisrow, kdst_p, kcode_p, gbuf, acc_sh, gsem, asem) = refs
    else:
        (g_hbm, row_hbm, col_hbm, p_hbm,
         st_row, st_col, krow, kdst, ssem,
         kisrow, kdst_p, gbuf, acc_sh, gsem, asem) = refs
        st_code = code_hbm = kcode = kcode_p = sl = s_hbm = None

    w = _worker_id()
    s_id = lax.axis_index("s")
    lo = w * RPT
    sbase = s_id * RPT
    CD = npar - 2

    zero16 = jnp.zeros((16,), jnp.float32)
    zero16i = jnp.zeros((16,), jnp.int32)

    @pl.loop(0, GB)
    def _zg(i):
        for t in range(DP // 16):
            gbuf[0, i, pl.ds(t * 16, 16)] = zero16

    if with_s:
        @pl.loop(0, RPT * SW // 16)
        def _zs(i):
            sl[pl.ds(i * 16, 16)] = zero16

    for r in range(RPT // GB):
        pltpu.sync_copy(gbuf.at[0], acc_sh.at[pl.ds(sbase + r * GB, GB)])
    _ztail = RPT - (RPT // GB) * GB
    if _ztail:
        pltpu.sync_copy(
            gbuf.at[0].at[pl.ds(0, _ztail)],
            acc_sh.at[pl.ds(sbase + (RPT // GB) * GB, _ztail)])

    @pl.loop(0, KL // 16)
    def _zk(q):
        krow[pl.ds(q * 16, 16)] = zero16i
        kdst[pl.ds(q * 16, 16)] = zero16i

    def stage_start(c, b):
        pltpu.make_async_copy(
            row_hbm.at[pl.ds(c * CE, CE)], st_row.at[b], ssem.at[b]).start()
        pltpu.make_async_copy(
            col_hbm.at[pl.ds(c * CE, CE)], st_col.at[b], ssem.at[b]).start()
        if with_s:
            pltpu.make_async_copy(
                code_hbm.at[pl.ds(c * CE, CE)], st_code.at[b], ssem.at[b]).start()

    def stage_wait(b):
        pltpu.make_async_copy(
            row_hbm.at[pl.ds(0, CE)], st_row.at[b], ssem.at[b]).wait()
        pltpu.make_async_copy(
            col_hbm.at[pl.ds(0, CE)], st_col.at[b], ssem.at[b]).wait()
        if with_s:
            pltpu.make_async_copy(
                code_hbm.at[pl.ds(0, CE)], st_code.at[b], ssem.at[b]).wait()

    def s_update(b):
        gb = gbuf.at[b]
        iota = lax.iota(jnp.int32, 16)

        @pl.loop(0, GB // 16)
        def _s_q(jq):
            dstv = kdst_p[b, pl.ds(jq * 16, 16)]
            cdv = kcode_p[b, pl.ds(jq * 16, 16)]
            jvec = jq * 16 + iota
            svals = plsc.load_gather(
                gb, [jvec, jnp.full((16,), D, jnp.int32)])
            sidx = (dstv - sbase) * SW + cdv
            for k in range(16):
                plsc.addupdate_scatter(sl, [sidx], svals, mask=iota == k)

    def add_start(b):
        pltpu.async_copy(gbuf.at[b], acc_sh.at[kdst_p.at[b]], asem.at[b],
                         add=True)

    def add_wait(b):
        pltpu.make_async_copy(gbuf.at[b], acc_sh.at[kdst_p.at[b]],
                              asem.at[b]).wait()

    def issue_batch(par, off):
        for q in range(GB // 16):
            s16 = pl.ds(off + q * 16, 16)
            d16 = pl.ds(q * 16, 16)
            kisrow[par, d16] = krow[s16]
            kdst_p[par, d16] = kdst[s16]
            if with_s:
                kcode_p[par, d16] = kcode[s16]
        pltpu.make_async_copy(
            g_hbm.at[kisrow.at[par]], gbuf.at[par], gsem.at[par]).start()

    def gather_wait(b):
        pltpu.make_async_copy(
            g_hbm.at[kisrow.at[b]], gbuf.at[b], gsem.at[b]).wait()

    def consume(b):
        gather_wait(b)
        if with_s:
            s_update(b)
        add_start(b)

    stage_start(0, 0)

    def chunk(c, carry):
        cursor, kglob = carry
        b = lax.rem(c, 2)
        stage_wait(b)

        @pl.when(c + 1 < NCHUNK)
        def _():
            stage_start(c + 1, 1 - b)

        for t in range(CE // 16):
            s16 = pl.ds(t * 16, 16)
            colv = st_col[b, s16]
            rowv = st_row[b, s16]
            m = (colv >= lo) & (colv < lo + RPT)
            plsc.store_compressed(krow.at[pl.ds(cursor, 16)], rowv, mask=m)
            plsc.store_compressed(
                kdst.at[pl.ds(cursor, 16)], colv - lo + sbase, mask=m)
            if with_s:
                plsc.store_compressed(
                    kcode.at[pl.ds(cursor, 16)], st_code[b, s16], mask=m)
            cursor = cursor + plsc.all_reduce_population_count(m)[0]

        nb = lax.div(cursor, GB)

        def batch(k, kg_c):
            kg = kg_c + k
            p = lax.rem(kg, npar)

            @pl.when(kg >= npar)
            def _():
                add_wait(p)

            issue_batch(p, k * GB)

            @pl.when(kg >= CD)
            def _():
                consume(lax.rem(kg - CD, npar))

            return kg_c

        lax.fori_loop(0, nb, batch, kglob)
        kglob = kglob + nb
        rem = cursor - nb * GB

        @pl.when(nb > 0)
        def _():
            for q in range(GB // 16):
                s16 = pl.ds(nb * GB + q * 16, 16)
                d16 = pl.ds(q * 16, 16)
                tmp_r = krow[s16]
                tmp_d = kdst[s16]
                if with_s:
                    tmp_c = kcode[s16]
                krow[d16] = tmp_r
                kdst[d16] = tmp_d
                if with_s:
                    kcode[d16] = tmp_c

        return (rem, kglob)

    cursor, kglob = lax.fori_loop(
        0, NCHUNK, chunk, (jnp.int32(0), jnp.int32(0)))

    @pl.when(cursor > 0)
    def _():
        p = lax.rem(kglob, npar)

        @pl.when(kglob >= npar)
        def _():
            add_wait(p)

        issue_batch(p, 0)

    ktot = kglob + jnp.where(cursor > 0, 1, 0).astype(jnp.int32)

    for d in range(CD + 1, 0, -1):
        cond = ktot >= d
        if d == CD + 1:
            cond = cond & (cursor > 0)

        @pl.when(cond)
        def _(d=d):
            q = lax.rem(ktot - d, npar)
            gather_wait(q)

            if d == 1:
                @pl.when(cursor > 0)
                def _():
                    gbp = gbuf.at[q]

                    @pl.loop(cursor, GB)
                    def _zpad(j):
                        for t in range(DP // 16):
                            gbp[j, pl.ds(t * 16, 16)] = zero16

                    @pl.loop(0, GB // 16)
                    def _zdst(qq):
                        iota2 = lax.iota(jnp.int32, 16)
                        s16 = pl.ds(qq * 16, 16)
                        keep = iota2 + qq * 16 < cursor
                        dv = kdst_p[q, s16]
                        kdst_p[q, s16] = jnp.where(keep, dv, sbase)
                        if with_s:
                            cv = kcode_p[q, s16]
                            kcode_p[q, s16] = jnp.where(keep, cv, 0)

            if with_s:
                s_update(q)
            add_start(q)

    for d in range(1, npar + 1):
        @pl.when(ktot >= d)
        def _(d=d):
            add_wait(lax.rem(ktot - d, npar))

    pltpu.sync_copy(acc_sh.at[pl.ds(sbase, RPT)], p_hbm.at[pl.ds(lo, RPT)])
    if with_s:
        pltpu.sync_copy(sl, s_hbm.at[pl.ds(lo * SW, RPT * SW)])


def _edges_call(g, row, col, code):
    with_s = code is not None
    npar = 4 if with_s else 6
    out_type = [jax.ShapeDtypeStruct((NP, DP), jnp.float32)]
    if with_s:
        out_type.append(jax.ShapeDtypeStruct((NP * SW,), jnp.float32))
    scratch = [
        pltpu.VMEM((2, CE), jnp.int32),
        pltpu.VMEM((2, CE), jnp.int32),
    ]
    if with_s:
        scratch.append(pltpu.VMEM((2, CE), jnp.int32))
    scratch += [
        pltpu.VMEM((KL,), jnp.int32),
        pltpu.VMEM((KL,), jnp.int32),
    ]
    if with_s:
        scratch.append(pltpu.VMEM((KL,), jnp.int32))
    if with_s:
        scratch.append(pltpu.VMEM((RPT * SW,), jnp.float32))
    scratch.append(pltpu.SemaphoreType.DMA((2,)))
    scratch += [
        pltpu.VMEM((npar, GB), jnp.int32),
        pltpu.VMEM((npar, GB), jnp.int32),
    ]
    if with_s:
        scratch.append(pltpu.VMEM((npar, GB), jnp.int32))
    scratch += [
        pltpu.VMEM((npar, GB, DP), jnp.float32),
        pltpu.VMEM_SHARED((NP // 2, DP), jnp.float32),
        pltpu.SemaphoreType.DMA((npar,)),
        pltpu.SemaphoreType.DMA((npar,)),
    ]
    f = pl.kernel(
        functools.partial(_edges_body, with_s, npar),
        out_type=tuple(out_type),
        mesh=_MESH,
        scratch_types=scratch,
        compiler_params=_SC_PARAMS,
    )
    if with_s:
        return f(g, row, col, code)
    return f(g, row, col)[0]


def _prep_body(hist_ref, x_ref, w_ref, b2_ref, g_ref, dis_ref):
    ones = jnp.ones((NT, 1), jnp.float32)
    deg = lax.dot_general(
        hist_ref[...], ones, (((0,), (0,)), ((), ())),
        preferred_element_type=jnp.float32) + 1.0
    dis = lax.rsqrt(deg)
    h = lax.dot_general(
        x_ref[...], w_ref[...], (((1,), (1,)), ((), ())),
        preferred_element_type=jnp.float32)
    g_ref[...] = (h + b2_ref[...]) * dis
    dis_ref[...] = dis


def _prep_call(hist, x, w0p, bias2):
    return pl.pallas_call(
        _prep_body,
        grid=(NP // BLK,),
        in_specs=[
            pl.BlockSpec((NT, BLK), lambda i: (0, i)),
            pl.BlockSpec((BLK, D), lambda i: (i, 0)),
            pl.BlockSpec((DP, D), lambda i: (0, 0)),
            pl.BlockSpec((1, DP), lambda i: (0, 0)),
        ],
        out_specs=[
            pl.BlockSpec((BLK, DP), lambda i: (i, 0)),
            pl.BlockSpec((BLK, 1), lambda i: (i, 0)),
        ],
        out_shape=[
            jax.ShapeDtypeStruct((NP, DP), jnp.float32),
            jax.ShapeDtypeStruct((NP, 1), jnp.float32),
        ],
    )(hist, x, w0p, bias2)


def _combine_body(with_matmul, p_ref, s_ref, g_ref, dis_ref, emb_ref, ts_ref,
                  *rest):
    if with_matmul:
        w_ref, b2_ref, out_ref = rest
    else:
        (out_ref,) = rest
    dis = dis_ref[...]
    se = lax.dot_general(
        s_ref[...], emb_ref[...], (((1,), (0,)), ((), ())),
        preferred_element_type=jnp.float32)
    pre = dis * (p_ref[...] + se + g_ref[...] + dis * ts_ref[...])
    if with_matmul:
        x1 = jnp.maximum(pre, 0.0)
        h = lax.dot_general(
            x1, w_ref[...], (((1,), (1,)), ((), ())),
            preferred_element_type=jnp.float32)
        out_ref[...] = dis * (h + b2_ref[...])
    else:
        out_ref[...] = pre


def _combine_call(p, s, g, dis, embp, tsp, w1p=None, bias2=None):
    with_matmul = w1p is not None
    in_specs = [
        pl.BlockSpec((BLK, DP), lambda i: (i, 0)),
        pl.BlockSpec((BLK, SW), lambda i: (i, 0)),
        pl.BlockSpec((BLK, DP), lambda i: (i, 0)),
        pl.BlockSpec((BLK, 1), lambda i: (i, 0)),
        pl.BlockSpec((SW, DP), lambda i: (0, 0)),
        pl.BlockSpec((1, DP), lambda i: (0, 0)),
    ]
    args = [p, s, g, dis, embp, tsp]
    if with_matmul:
        in_specs += [
            pl.BlockSpec((DP, DP), lambda i: (0, 0)),
            pl.BlockSpec((1, DP), lambda i: (0, 0)),
        ]
        args += [w1p, bias2]
    return pl.pallas_call(
        functools.partial(_combine_body, with_matmul),
        grid=(NP // BLK,),
        in_specs=in_specs,
        out_specs=pl.BlockSpec((BLK, DP), lambda i: (i, 0)),
        out_shape=jax.ShapeDtypeStruct((NP, DP), jnp.float32),
    )(*args)


def _pad_tables(e1, e2, b):
    emb = (e1[:3][:, None, :] + e2[None, :3, :]).reshape(9, D)
    embp = jnp.zeros((SW, DP), jnp.float32).at[:9, :D].set(emb)
    tsp = jnp.zeros((1, DP), jnp.float32).at[0, :D].set(e1[4] + e2[0])
    bias2 = jnp.zeros((1, DP), jnp.float32).at[0, :D].set(b).at[0, D].set(1.0)
    return embp, tsp, bias2


def kernel(x, edge_index, edge_attr, W0, b0, e1_0, e2_0, W1, b1, e1_1, e2_1):
    row = edge_index[0]
    col = edge_index[1]
    code = edge_attr[:, 0] * 3 + edge_attr[:, 1]

    emb0p, ts0p, bias2_0 = _pad_tables(e1_0, e2_0, b0)
    emb1p, ts1p, bias2_1 = _pad_tables(e1_1, e2_1, b1)
    w0p = jnp.zeros((DP, D), jnp.float32).at[:D].set(W0)
    w1p = jnp.zeros((DP, DP), jnp.float32).at[:D, :D].set(W1)

    hist = _hist_call(row)
    g0p, dis = _prep_call(hist, x, w0p, bias2_0)
    p0, s = _edges_call(g0p, row, col, code)
    s = s.reshape(NP, SW)
    g1p = _combine_call(p0, s, g0p, dis, emb0p, ts0p, w1p, bias2_1)
    p1 = _edges_call(g1p, row, col, None)
    outp = _combine_call(p1, s, g1p, dis, emb1p, ts1p)
    return outp[:N, :D]


# speedup vs baseline: 1.3102x; 1.3102x over previous
import functools

import jax
import jax.numpy as jnp
from jax import lax
from jax.experimental import pallas as pl
from jax.experimental.pallas import tpu as pltpu
from jax.experimental.pallas import tpu_sc as plsc

N = 10000
E = 160000
D = 300
DP = 304
NP = 10240
NT = 32
RPT = 320
CE = 640
NCHUNK = E // CE
GB = 32
KL = 704
SW = 16
EPT = E // NT
BLK = 1024

_MESH = plsc.VectorSubcoreMesh(core_axis_name="c", subcore_axis_name="s")
_SC_PARAMS = pltpu.CompilerParams(
    needs_layout_passes=False, use_tc_tiling_on_sc=False)


def _worker_id():
    return lax.axis_index("s") * 2 + lax.axis_index("c")


def _hist_body(row_hbm, hist_hbm, hl, st):
    w = _worker_id()
    zero16 = jnp.zeros((16,), jnp.float32)
    ones16 = jnp.ones((16,), jnp.float32)
    iota = lax.iota(jnp.int32, 16)

    @pl.loop(0, NP // 16)
    def _zero(i):
        hl[pl.ds(i * 16, 16)] = zero16

    base = w * EPT
    pltpu.sync_copy(row_hbm.at[pl.ds(base, EPT)], st)

    nfull = EPT // 16

    def step(i, carry):
        idxv = st[pl.ds(i * 16, 16)]
        for k in range(16):
            plsc.addupdate_scatter(hl, [idxv], ones16, mask=iota == k)
        return carry

    lax.fori_loop(0, nfull, step, 0)
    tailv = st[pl.ds(EPT - 16, 16)]
    for k in range(16 - (EPT - nfull * 16), 16):
        plsc.addupdate_scatter(hl, [tailv], ones16, mask=iota == k)
    pltpu.sync_copy(hl, hist_hbm.at[w])


def _hist_call(row):
    f = pl.kernel(
        _hist_body,
        out_type=jax.ShapeDtypeStruct((NT, NP), jnp.float32),
        mesh=_MESH,
        scratch_types=[
            pltpu.VMEM((NP,), jnp.float32),
            pltpu.VMEM((EPT,), jnp.int32),
        ],
        compiler_params=_SC_PARAMS,
    )
    return f(row)


def _edges_body(with_s, npar, cd, *refs):
    if with_s:
        (g_hbm, row_hbm, col_hbm, code_hbm, p_hbm, s_hbm,
         st_row, st_col, st_code, krow, kdst, kcode, sl, ssem,
         kisrow, kdst_p, kcode_p, gbuf, acc_sh, gsem, asem) = refs
    else:
        (g_hbm, row_hbm, col_hbm, p_hbm,
         st_row, st_col, krow, kdst, ssem,
         kisrow, kdst_p, gbuf, acc_sh, gsem, asem) = refs
        st_code = code_hbm = kcode = kcode_p = sl = s_hbm = None

    w = _worker_id()
    s_id = lax.axis_index("s")
    lo = w * RPT
    sbase = s_id * RPT
    CD = cd

    zero16 = jnp.zeros((16,), jnp.float32)
    zero16i = jnp.zeros((16,), jnp.int32)

    @pl.loop(0, GB)
    def _zg(i):
        for t in range(DP // 16):
            gbuf[0, i, pl.ds(t * 16, 16)] = zero16

    if with_s:
        @pl.loop(0, RPT * SW // 16)
        def _zs(i):
            sl[pl.ds(i * 16, 16)] = zero16

    for r in range(RPT // GB):
        pltpu.sync_copy(gbuf.at[0], acc_sh.at[pl.ds(sbase + r * GB, GB)])
    _ztail = RPT - (RPT // GB) * GB
    if _ztail:
        pltpu.sync_copy(
            gbuf.at[0].at[pl.ds(0, _ztail)],
            acc_sh.at[pl.ds(sbase + (RPT // GB) * GB, _ztail)])

    @pl.loop(0, KL // 16)
    def _zk(q):
        krow[pl.ds(q * 16, 16)] = zero16i
        kdst[pl.ds(q * 16, 16)] = zero16i

    def stage_start(c, b):
        pltpu.make_async_copy(
            row_hbm.at[pl.ds(c * CE, CE)], st_row.at[b], ssem.at[b]).start()
        pltpu.make_async_copy(
            col_hbm.at[pl.ds(c * CE, CE)], st_col.at[b], ssem.at[b]).start()
        if with_s:
            pltpu.make_async_copy(
                code_hbm.at[pl.ds(c * CE, CE)], st_code.at[b], ssem.at[b]).start()

    def stage_wait(b):
        pltpu.make_async_copy(
            row_hbm.at[pl.ds(0, CE)], st_row.at[b], ssem.at[b]).wait()
        pltpu.make_async_copy(
            col_hbm.at[pl.ds(0, CE)], st_col.at[b], ssem.at[b]).wait()
        if with_s:
            pltpu.make_async_copy(
                code_hbm.at[pl.ds(0, CE)], st_code.at[b], ssem.at[b]).wait()

    def s_update(b):
        gb = gbuf.at[b]
        iota = lax.iota(jnp.int32, 16)

        @pl.loop(0, GB // 16)
        def _s_q(jq):
            dstv = kdst_p[b, pl.ds(jq * 16, 16)]
            cdv = kcode_p[b, pl.ds(jq * 16, 16)]
            jvec = jq * 16 + iota
            svals = plsc.load_gather(
                gb, [jvec, jnp.full((16,), D, jnp.int32)])
            sidx = (dstv - sbase) * SW + cdv
            for k in range(16):
                plsc.addupdate_scatter(sl, [sidx], svals, mask=iota == k)

    def add_start(b):
        pltpu.async_copy(gbuf.at[b], acc_sh.at[kdst_p.at[b]], asem.at[b],
                         add=True)

    def add_wait(b):
        pltpu.make_async_copy(gbuf.at[b], acc_sh.at[kdst_p.at[b]],
                              asem.at[b]).wait()

    def issue_batch(par, off):
        for q in range(GB // 16):
            s16 = pl.ds(off + q * 16, 16)
            d16 = pl.ds(q * 16, 16)
            kisrow[par, d16] = krow[s16]
            kdst_p[par, d16] = kdst[s16]
            if with_s:
                kcode_p[par, d16] = kcode[s16]
        pltpu.make_async_copy(
            g_hbm.at[kisrow.at[par]], gbuf.at[par], gsem.at[par]).start()

    def gather_wait(b):
        pltpu.make_async_copy(
            g_hbm.at[kisrow.at[b]], gbuf.at[b], gsem.at[b]).wait()

    def consume(b):
        gather_wait(b)
        if with_s:
            s_update(b)
        add_start(b)

    stage_start(0, 0)

    def chunk(c, carry):
        cursor, kglob = carry
        b = lax.rem(c, 2)
        stage_wait(b)

        @pl.when(c + 1 < NCHUNK)
        def _():
            stage_start(c + 1, 1 - b)

        for t in range(CE // 16):
            s16 = pl.ds(t * 16, 16)
            colv = st_col[b, s16]
            rowv = st_row[b, s16]
            m = (colv >= lo) & (colv < lo + RPT)
            plsc.store_compressed(krow.at[pl.ds(cursor, 16)], rowv, mask=m)
            plsc.store_compressed(
                kdst.at[pl.ds(cursor, 16)], colv - lo + sbase, mask=m)
            if with_s:
                plsc.store_compressed(
                    kcode.at[pl.ds(cursor, 16)], st_code[b, s16], mask=m)
            cursor = cursor + plsc.all_reduce_population_count(m)[0]

        nb = lax.div(cursor, GB)

        def batch(k, kg_c):
            kg = kg_c + k
            p = lax.rem(kg, npar)

            @pl.when(kg >= npar)
            def _():
                add_wait(p)

            issue_batch(p, k * GB)

            @pl.when(kg >= CD)
            def _():
                consume(lax.rem(kg - CD, npar))

            return kg_c

        lax.fori_loop(0, nb, batch, kglob)
        kglob = kglob + nb
        rem = cursor - nb * GB

        @pl.when(nb > 0)
        def _():
            for q in range(GB // 16):
                s16 = pl.ds(nb * GB + q * 16, 16)
                d16 = pl.ds(q * 16, 16)
                tmp_r = krow[s16]
                tmp_d = kdst[s16]
                if with_s:
                    tmp_c = kcode[s16]
                krow[d16] = tmp_r
                kdst[d16] = tmp_d
                if with_s:
                    kcode[d16] = tmp_c

        return (rem, kglob)

    cursor, kglob = lax.fori_loop(
        0, NCHUNK, chunk, (jnp.int32(0), jnp.int32(0)))

    @pl.when(cursor > 0)
    def _():
        p = lax.rem(kglob, npar)

        @pl.when(kglob >= npar)
        def _():
            add_wait(p)

        issue_batch(p, 0)

    ktot = kglob + jnp.where(cursor > 0, 1, 0).astype(jnp.int32)

    for d in range(CD + 1, 0, -1):
        cond = ktot >= d
        if d == CD + 1:
            cond = cond & (cursor > 0)

        @pl.when(cond)
        def _(d=d):
            q = lax.rem(ktot - d, npar)
            gather_wait(q)

            if d == 1:
                @pl.when(cursor > 0)
                def _():
                    gbp = gbuf.at[q]

                    @pl.loop(cursor, GB)
                    def _zpad(j):
                        for t in range(DP // 16):
                            gbp[j, pl.ds(t * 16, 16)] = zero16

                    @pl.loop(0, GB // 16)
                    def _zdst(qq):
                        iota2 = lax.iota(jnp.int32, 16)
                        s16 = pl.ds(qq * 16, 16)
                        keep = iota2 + qq * 16 < cursor
                        dv = kdst_p[q, s16]
                        kdst_p[q, s16] = jnp.where(keep, dv, sbase)
                        if with_s:
                            cv = kcode_p[q, s16]
                            kcode_p[q, s16] = jnp.where(keep, cv, 0)

            if with_s:
                s_update(q)
            add_start(q)

    for d in range(1, npar + 1):
        @pl.when(ktot >= d)
        def _(d=d):
            add_wait(lax.rem(ktot - d, npar))

    pltpu.sync_copy(acc_sh.at[pl.ds(sbase, RPT)], p_hbm.at[pl.ds(lo, RPT)])
    if with_s:
        pltpu.sync_copy(sl, s_hbm.at[pl.ds(lo * SW, RPT * SW)])


def _edges_call(g, row, col, code):
    with_s = code is not None
    npar, cd = (2, 1) if with_s else (3, 2)
    out_type = [jax.ShapeDtypeStruct((NP, DP), jnp.float32)]
    if with_s:
        out_type.append(jax.ShapeDtypeStruct((NP * SW,), jnp.float32))
    scratch = [
        pltpu.VMEM((2, CE), jnp.int32),
        pltpu.VMEM((2, CE), jnp.int32),
    ]
    if with_s:
        scratch.append(pltpu.VMEM((2, CE), jnp.int32))
    scratch += [
        pltpu.VMEM((KL,), jnp.int32),
        pltpu.VMEM((KL,), jnp.int32),
    ]
    if with_s:
        scratch.append(pltpu.VMEM((KL,), jnp.int32))
    if with_s:
        scratch.append(pltpu.VMEM((RPT * SW,), jnp.float32))
    scratch.append(pltpu.SemaphoreType.DMA((2,)))
    scratch += [
        pltpu.VMEM((npar, GB), jnp.int32),
        pltpu.VMEM((npar, GB), jnp.int32),
    ]
    if with_s:
        scratch.append(pltpu.VMEM((npar, GB), jnp.int32))
    scratch += [
        pltpu.VMEM((npar, GB, DP), jnp.float32),
        pltpu.VMEM_SHARED((NP // 2, DP), jnp.float32),
        pltpu.SemaphoreType.DMA((npar,)),
        pltpu.SemaphoreType.DMA((npar,)),
    ]
    f = pl.kernel(
        functools.partial(_edges_body, with_s, npar, cd),
        out_type=tuple(out_type),
        mesh=_MESH,
        scratch_types=scratch,
        compiler_params=_SC_PARAMS,
    )
    if with_s:
        return f(g, row, col, code)
    return f(g, row, col)[0]


def _prep_body(hist_ref, x_ref, w_ref, b2_ref, g_ref, dis_ref):
    ones = jnp.ones((NT, 1), jnp.float32)
    deg = lax.dot_general(
        hist_ref[...], ones, (((0,), (0,)), ((), ())),
        preferred_element_type=jnp.float32) + 1.0
    dis = lax.rsqrt(deg)
    h = lax.dot_general(
        x_ref[...], w_ref[...], (((1,), (1,)), ((), ())),
        preferred_element_type=jnp.float32)
    g_ref[...] = (h + b2_ref[...]) * dis
    dis_ref[...] = dis


def _prep_call(hist, x, w0p, bias2):
    return pl.pallas_call(
        _prep_body,
        grid=(NP // BLK,),
        in_specs=[
            pl.BlockSpec((NT, BLK), lambda i: (0, i)),
            pl.BlockSpec((BLK, D), lambda i: (i, 0)),
            pl.BlockSpec((DP, D), lambda i: (0, 0)),
            pl.BlockSpec((1, DP), lambda i: (0, 0)),
        ],
        out_specs=[
            pl.BlockSpec((BLK, DP), lambda i: (i, 0)),
            pl.BlockSpec((BLK, 1), lambda i: (i, 0)),
        ],
        out_shape=[
            jax.ShapeDtypeStruct((NP, DP), jnp.float32),
            jax.ShapeDtypeStruct((NP, 1), jnp.float32),
        ],
    )(hist, x, w0p, bias2)


def _combine_body(with_matmul, p_ref, s_ref, g_ref, dis_ref, emb_ref, ts_ref,
                  *rest):
    if with_matmul:
        w_ref, b2_ref, out_ref = rest
    else:
        (out_ref,) = rest
    dis = dis_ref[...]
    se = lax.dot_general(
        s_ref[...], emb_ref[...], (((1,), (0,)), ((), ())),
        preferred_element_type=jnp.float32)
    pre = dis * (p_ref[...] + se + g_ref[...] + dis * ts_ref[...])
    if with_matmul:
        x1 = jnp.maximum(pre, 0.0)
        h = lax.dot_general(
            x1, w_ref[...], (((1,), (1,)), ((), ())),
            preferred_element_type=jnp.float32)
        out_ref[...] = dis * (h + b2_ref[...])
    else:
        out_ref[...] = pre


def _combine_call(p, s, g, dis, embp, tsp, w1p=None, bias2=None):
    with_matmul = w1p is not None
    in_specs = [
        pl.BlockSpec((BLK, DP), lambda i: (i, 0)),
        pl.BlockSpec((BLK, SW), lambda i: (i, 0)),
        pl.BlockSpec((BLK, DP), lambda i: (i, 0)),
        pl.BlockSpec((BLK, 1), lambda i: (i, 0)),
        pl.BlockSpec((SW, DP), lambda i: (0, 0)),
        pl.BlockSpec((1, DP), lambda i: (0, 0)),
    ]
    args = [p, s, g, dis, embp, tsp]
    if with_matmul:
        in_specs += [
            pl.BlockSpec((DP, DP), lambda i: (0, 0)),
            pl.BlockSpec((1, DP), lambda i: (0, 0)),
        ]
        args += [w1p, bias2]
    return pl.pallas_call(
        functools.partial(_combine_body, with_matmul),
        grid=(NP // BLK,),
        in_specs=in_specs,
        out_specs=pl.BlockSpec((BLK, DP), lambda i: (i, 0)),
        out_shape=jax.ShapeDtypeStruct((NP, DP), jnp.float32),
    )(*args)


def _pad_tables(e1, e2, b):
    emb = (e1[:3][:, None, :] + e2[None, :3, :]).reshape(9, D)
    embp = jnp.zeros((SW, DP), jnp.float32).at[:9, :D].set(emb)
    tsp = jnp.zeros((1, DP), jnp.float32).at[0, :D].set(e1[4] + e2[0])
    bias2 = jnp.zeros((1, DP), jnp.float32).at[0, :D].set(b).at[0, D].set(1.0)
    return embp, tsp, bias2


def kernel(x, edge_index, edge_attr, W0, b0, e1_0, e2_0, W1, b1, e1_1, e2_1):
    row = edge_index[0]
    col = edge_index[1]
    code = edge_attr[:, 0] * 3 + edge_attr[:, 1]

    emb0p, ts0p, bias2_0 = _pad_tables(e1_0, e2_0, b0)
    emb1p, ts1p, bias2_1 = _pad_tables(e1_1, e2_1, b1)
    w0p = jnp.zeros((DP, D), jnp.float32).at[:D].set(W0)
    w1p = jnp.zeros((DP, DP), jnp.float32).at[:D, :D].set(W1)

    hist = _hist_call(row)
    g0p, dis = _prep_call(hist, x, w0p, bias2_0)
    p0, s = _edges_call(g0p, row, col, code)
    s = s.reshape(NP, SW)
    g1p = _combine_call(p0, s, g0p, dis, emb0p, ts0p, w1p, bias2_1)
    p1 = _edges_call(g1p, row, col, None)
    outp = _combine_call(p1, s, g1p, dis, emb1p, ts1p)
    return outp[:N, :D]
